# baseline (speedup 1.0000x reference)
import functools

import jax
import jax.numpy as jnp
from jax import lax
from jax.experimental import pallas as pl
from jax.experimental.pallas import tpu as pltpu
from jax.experimental.pallas import tpu_sc as plsc

N_NODES = 10000
N_EDGES = 320000
D_FEAT = 128

NUM_WORKERS = 32
EDGES_PER_WORKER = N_EDGES // NUM_WORKERS
CHUNK = 80
NCHUNKS = EDGES_PER_WORKER // CHUNK
GROUPS = CHUNK // 16
LANES = 16


def _edge_dot_kernel(emb_hbm, src_hbm, dst_hbm, out_hbm,
                     idx_src, idx_dst, rows_src, rows_dst, out_buf,
                     sem_a0, sem_b0, sem_a1, sem_b1):
    wid = lax.axis_index("s") * 2 + lax.axis_index("c")
    tile_base = wid * EDGES_PER_WORKER

    pltpu.sync_copy(src_hbm.at[pl.ds(tile_base, EDGES_PER_WORKER)], idx_src)
    pltpu.sync_copy(dst_hbm.at[pl.ds(tile_base, EDGES_PER_WORKER)], idx_dst)

    sems = ((sem_a0, sem_b0), (sem_a1, sem_b1))

    def start(c, slot):
        sa, sb = sems[slot]
        pltpu.make_async_copy(
            emb_hbm.at[idx_src.at[pl.ds(c * CHUNK, CHUNK)]],
            rows_src.at[slot], sa).start()
        pltpu.make_async_copy(
            emb_hbm.at[idx_dst.at[pl.ds(c * CHUNK, CHUNK)]],
            rows_dst.at[slot], sb).start()

    def wait(c, slot):
        sa, sb = sems[slot]
        pltpu.make_async_copy(
            emb_hbm.at[idx_src.at[pl.ds(c * CHUNK, CHUNK)]],
            rows_src.at[slot], sa).wait()
        pltpu.make_async_copy(
            emb_hbm.at[idx_dst.at[pl.ds(c * CHUNK, CHUNK)]],
            rows_dst.at[slot], sb).wait()

    def compute(c, slot):
        lane = lax.iota(jnp.int32, LANES)

        @plsc.parallel_loop(0, GROUPS, 1, unroll=2)
        def group_body(g):
            res = jnp.zeros((LANES,), jnp.float32)
            for j in range(LANES):
                e = g * LANES + j
                accs = [jnp.zeros((LANES,), jnp.float32) for _ in range(4)]
                for k in range(D_FEAT // LANES):
                    a = rows_src[slot, e, pl.ds(k * LANES, LANES)]
                    b = rows_dst[slot, e, pl.ds(k * LANES, LANES)]
                    accs[k % 4] = accs[k % 4] + a * b
                acc = (accs[0] + accs[1]) + (accs[2] + accs[3])
                tot = plsc.cumsum(acc)
                tot = lax.gather(
                    tot, jnp.full((LANES, 1), LANES - 1, jnp.int32),
                    lax.GatherDimensionNumbers(
                        offset_dims=(), collapsed_slice_dims=(0,),
                        start_index_map=(0,)),
                    (1,), mode=lax.GatherScatterMode.PROMISE_IN_BOUNDS)
                res = jnp.where(lane == j, tot, res)
            out_buf[pl.ds(c * CHUNK + g * LANES, LANES)] = res

    start(0, 0)
    start(1, 1)

    def pair_body(i, carry):
        c0 = 2 * i
        wait(c0, 0)
        compute(c0, 0)
        start(c0 + 2, 0)
        wait(c0 + 1, 1)
        compute(c0 + 1, 1)

        @pl.when(c0 + 3 < NCHUNKS)
        def _():
            start(c0 + 3, 1)

        return carry

    lax.fori_loop(0, (NCHUNKS - 1) // 2, pair_body, 0)

    last = NCHUNKS - 1
    wait(last, 0)
    compute(last, 0)

    pltpu.sync_copy(out_buf, out_hbm.at[pl.ds(tile_base, EDGES_PER_WORKER)])


@jax.jit
def kernel(embeddings, src, dst):
    mesh = plsc.VectorSubcoreMesh(core_axis_name="c", subcore_axis_name="s")
    k = functools.partial(
        pl.kernel,
        mesh=mesh,
        out_type=jax.ShapeDtypeStruct((N_EDGES,), jnp.float32),
        scratch_types=[
            pltpu.VMEM((EDGES_PER_WORKER,), jnp.int32),
            pltpu.VMEM((EDGES_PER_WORKER,), jnp.int32),
            pltpu.VMEM((2, CHUNK, D_FEAT), jnp.float32),
            pltpu.VMEM((2, CHUNK, D_FEAT), jnp.float32),
            pltpu.VMEM((EDGES_PER_WORKER,), jnp.float32),
            pltpu.SemaphoreType.DMA,
            pltpu.SemaphoreType.DMA,
            pltpu.SemaphoreType.DMA,
            pltpu.SemaphoreType.DMA,
        ],
        compiler_params=pltpu.CompilerParams(needs_layout_passes=False),
    )(_edge_dot_kernel)
    return k(embeddings, src, dst)


# speedup vs baseline: 1.3472x; 1.3472x over previous
import functools

import jax
import jax.numpy as jnp
from jax import lax
from jax.experimental import pallas as pl
from jax.experimental.pallas import tpu as pltpu
from jax.experimental.pallas import tpu_sc as plsc

N_NODES = 10000
N_EDGES = 320000
D_FEAT = 128

NUM_WORKERS = 32
EDGES_PER_WORKER = N_EDGES // NUM_WORKERS
CHUNK = 80
NCHUNKS = EDGES_PER_WORKER // CHUNK
GROUPS = CHUNK // 16
LANES = 16


def _edge_dot_kernel(emb_hbm, src_hbm, dst_hbm, out_hbm,
                     idx_src, idx_dst, rows_src, rows_dst, out_buf,
                     sem_a0, sem_b0, sem_a1, sem_b1):
    wid = lax.axis_index("s") * 2 + lax.axis_index("c")
    tile_base = wid * EDGES_PER_WORKER

    pltpu.sync_copy(src_hbm.at[pl.ds(tile_base, EDGES_PER_WORKER)], idx_src)
    pltpu.sync_copy(dst_hbm.at[pl.ds(tile_base, EDGES_PER_WORKER)], idx_dst)

    sems = ((sem_a0, sem_b0), (sem_a1, sem_b1))

    def start(c, slot):
        sa, sb = sems[slot]
        pltpu.make_async_copy(
            emb_hbm.at[idx_src.at[pl.ds(c * CHUNK, CHUNK)]],
            rows_src.at[slot], sa).start()
        pltpu.make_async_copy(
            emb_hbm.at[idx_dst.at[pl.ds(c * CHUNK, CHUNK)]],
            rows_dst.at[slot], sb).start()

    def wait(c, slot):
        sa, sb = sems[slot]
        pltpu.make_async_copy(
            emb_hbm.at[idx_src.at[pl.ds(c * CHUNK, CHUNK)]],
            rows_src.at[slot], sa).wait()
        pltpu.make_async_copy(
            emb_hbm.at[idx_dst.at[pl.ds(c * CHUNK, CHUNK)]],
            rows_dst.at[slot], sb).wait()

    def compute(c, slot):
        lane = lax.iota(jnp.int32, LANES)

        def group_body(g, carry):
            parts = []
            for j in range(LANES):
                e = g * LANES + j
                accs = [jnp.zeros((LANES,), jnp.float32) for _ in range(4)]
                for k in range(D_FEAT // LANES):
                    a = rows_src[slot, e, pl.ds(k * LANES, LANES)]
                    b = rows_dst[slot, e, pl.ds(k * LANES, LANES)]
                    accs[k % 4] = accs[k % 4] + a * b
                acc = (accs[0] + accs[1]) + (accs[2] + accs[3])
                tot = plsc.cumsum(acc)
                tot = lax.gather(
                    tot, jnp.full((LANES, 1), LANES - 1, jnp.int32),
                    lax.GatherDimensionNumbers(
                        offset_dims=(), collapsed_slice_dims=(0,),
                        start_index_map=(0,)),
                    (1,), mode=lax.GatherScatterMode.PROMISE_IN_BOUNDS)
                parts.append(jnp.where(lane == j, tot, 0.0))
            while len(parts) > 1:
                parts = [parts[i] + parts[i + 1]
                         for i in range(0, len(parts), 2)]
            out_buf[pl.ds(c * CHUNK + g * LANES, LANES)] = parts[0]
            return carry

        lax.fori_loop(0, GROUPS, group_body, 0)

    start(0, 0)
    start(1, 1)

    def pair_body(i, carry):
        c0 = 2 * i
        wait(c0, 0)
        compute(c0, 0)
        start(c0 + 2, 0)
        wait(c0 + 1, 1)
        compute(c0 + 1, 1)

        @pl.when(c0 + 3 < NCHUNKS)
        def _():
            start(c0 + 3, 1)

        return carry

    lax.fori_loop(0, (NCHUNKS - 1) // 2, pair_body, 0)

    last = NCHUNKS - 1
    wait(last, 0)
    compute(last, 0)

    pltpu.sync_copy(out_buf, out_hbm.at[pl.ds(tile_base, EDGES_PER_WORKER)])


@jax.jit
def kernel(embeddings, src, dst):
    mesh = plsc.VectorSubcoreMesh(core_axis_name="c", subcore_axis_name="s")
    k = functools.partial(
        pl.kernel,
        mesh=mesh,
        out_type=jax.ShapeDtypeStruct((N_EDGES,), jnp.float32),
        scratch_types=[
            pltpu.VMEM((EDGES_PER_WORKER,), jnp.int32),
            pltpu.VMEM((EDGES_PER_WORKER,), jnp.int32),
            pltpu.VMEM((2, CHUNK, D_FEAT), jnp.float32),
            pltpu.VMEM((2, CHUNK, D_FEAT), jnp.float32),
            pltpu.VMEM((EDGES_PER_WORKER,), jnp.float32),
            pltpu.SemaphoreType.DMA,
            pltpu.SemaphoreType.DMA,
            pltpu.SemaphoreType.DMA,
            pltpu.SemaphoreType.DMA,
        ],
        compiler_params=pltpu.CompilerParams(needs_layout_passes=False),
    )(_edge_dot_kernel)
    return k(embeddings, src, dst)


# speedup vs baseline: 2.2072x; 1.6384x over previous
import functools

import jax
import jax.numpy as jnp
from jax import lax
from jax.experimental import pallas as pl
from jax.experimental.pallas import tpu as pltpu
from jax.experimental.pallas import tpu_sc as plsc

N_NODES = 10000
N_EDGES = 320000
D_FEAT = 128

NUM_WORKERS = 32
EDGES_PER_WORKER = N_EDGES // NUM_WORKERS
CHUNK = 80
NCHUNKS = EDGES_PER_WORKER // CHUNK
GROUPS = CHUNK // 16
LANES = 16
NSLOTS = 3


def _norms_body(emb_ref, out_ref):
    x = emb_ref[...]
    out_ref[...] = jnp.sum(x * x, axis=1)


def _node_norms(embeddings):
    return pl.pallas_call(
        _norms_body,
        out_shape=jax.ShapeDtypeStruct((N_NODES,), jnp.float32),
    )(embeddings)


def _edge_dot_kernel(emb_hbm, src_hbm, dst_hbm, norms_hbm, out_hbm,
                     idx_src, idx_dst, rows, out_buf, norms_t,
                     sa0, sa1, sa2, sb0, sb1, sb2):
    wid = lax.axis_index("s") * 2 + lax.axis_index("c")
    tile_base = wid * EDGES_PER_WORKER

    pltpu.sync_copy(src_hbm.at[pl.ds(tile_base, EDGES_PER_WORKER)], idx_src)
    pltpu.sync_copy(dst_hbm.at[pl.ds(tile_base, EDGES_PER_WORKER)], idx_dst)
    pltpu.sync_copy(norms_hbm, norms_t)

    sems_a = (sa0, sa1, sa2)
    sems_b = (sb0, sb1, sb2)

    def start_a(c, s):
        pltpu.async_copy(
            emb_hbm.at[idx_src.at[pl.ds(c * CHUNK, CHUNK)]],
            rows.at[s], sems_a[s])

    def wait_a(c, s):
        pltpu.make_async_copy(
            emb_hbm.at[idx_src.at[pl.ds(c * CHUNK, CHUNK)]],
            rows.at[s], sems_a[s]).wait()

    def start_b(c, s):
        pltpu.async_copy(
            emb_hbm.at[idx_dst.at[pl.ds(c * CHUNK, CHUNK)]],
            rows.at[s], sems_b[s], add=True)

    def wait_b(c, s):
        pltpu.make_async_copy(
            emb_hbm.at[idx_dst.at[pl.ds(c * CHUNK, CHUNK)]],
            rows.at[s], sems_b[s]).wait()

    lane = lax.iota(jnp.int32, LANES)

    def compute(c, s):
        def group_body(g, carry):
            parts = []
            for j in range(LANES):
                e = g * LANES + j
                accs = [jnp.zeros((LANES,), jnp.float32) for _ in range(4)]
                for k in range(D_FEAT // LANES):
                    v = rows[s, e, pl.ds(k * LANES, LANES)]
                    accs[k % 4] = accs[k % 4] + v * v
                acc = (accs[0] + accs[1]) + (accs[2] + accs[3])
                tot = plsc.cumsum(acc)
                tot = lax.gather(
                    tot, jnp.full((LANES, 1), LANES - 1, jnp.int32),
                    lax.GatherDimensionNumbers(
                        offset_dims=(), collapsed_slice_dims=(0,),
                        start_index_map=(0,)),
                    (1,), mode=lax.GatherScatterMode.PROMISE_IN_BOUNDS)
                parts.append(jnp.where(lane == j, tot, 0.0))
            while len(parts) > 1:
                parts = [parts[i] + parts[i + 1]
                         for i in range(0, len(parts), 2)]
            ids_a = idx_src[pl.ds(c * CHUNK + g * LANES, LANES)]
            ids_b = idx_dst[pl.ds(c * CHUNK + g * LANES, LANES)]
            na = plsc.load_gather(norms_t, [ids_a])
            nb = plsc.load_gather(norms_t, [ids_b])
            out_buf[pl.ds(c * CHUNK + g * LANES, LANES)] = (
                0.5 * (parts[0] - na - nb))
            return carry

        lax.fori_loop(0, GROUPS, group_body, 0)

    def step(c, s):
        wait_b(c, s)
        wait_a(c + 1, (s + 1) % NSLOTS)
        start_b(c + 1, (s + 1) % NSLOTS)
        compute(c, s)

        @pl.when(c + 3 < NCHUNKS)
        def _():
            start_a(c + 3, s)

    start_a(0, 0)
    start_a(1, 1)
    start_a(2, 2)
    wait_a(0, 0)
    start_b(0, 0)

    def tri_body(i, carry):
        c0 = 3 * i
        step(c0, 0)
        step(c0 + 1, 1)
        step(c0 + 2, 2)
        return carry

    lax.fori_loop(0, (NCHUNKS - 2) // 3, tri_body, 0)

    wait_b(NCHUNKS - 2, 0)
    wait_a(NCHUNKS - 1, 1)
    start_b(NCHUNKS - 1, 1)
    compute(NCHUNKS - 2, 0)
    wait_b(NCHUNKS - 1, 1)
    compute(NCHUNKS - 1, 1)

    pltpu.sync_copy(out_buf, out_hbm.at[pl.ds(tile_base, EDGES_PER_WORKER)])


@jax.jit
def kernel(embeddings, src, dst):
    norms = _node_norms(embeddings)
    mesh = plsc.VectorSubcoreMesh(core_axis_name="c", subcore_axis_name="s")
    k = functools.partial(
        pl.kernel,
        mesh=mesh,
        out_type=jax.ShapeDtypeStruct((N_EDGES,), jnp.float32),
        scratch_types=[
            pltpu.VMEM((EDGES_PER_WORKER,), jnp.int32),
            pltpu.VMEM((EDGES_PER_WORKER,), jnp.int32),
            pltpu.VMEM((NSLOTS, CHUNK, D_FEAT), jnp.float32),
            pltpu.VMEM((EDGES_PER_WORKER,), jnp.float32),
            pltpu.VMEM((N_NODES,), jnp.float32),
            pltpu.SemaphoreType.DMA,
            pltpu.SemaphoreType.DMA,
            pltpu.SemaphoreType.DMA,
            pltpu.SemaphoreType.DMA,
            pltpu.SemaphoreType.DMA,
            pltpu.SemaphoreType.DMA,
        ],
        compiler_params=pltpu.CompilerParams(needs_layout_passes=False),
    )(_edge_dot_kernel)
    return k(embeddings, src, dst, norms)
